# baseline (speedup 1.0000x reference)
import functools
import math

import jax
import jax.numpy as jnp
from jax import lax
from jax.experimental import pallas as pl
from jax.experimental.pallas import tpu as pltpu
from jax.experimental.pallas import tpu_sc as plsc

D_MODEL = 1024
SCALE = math.sqrt(D_MODEL)

NUM_CORES = 2
NUM_SUBCORES = 16
LANES = 16
NUM_WORKERS = NUM_CORES * NUM_SUBCORES

CHUNK = 8
NBUF = 4


def _sc_embed(x2d, table):
    n_seq, seq_len = x2d.shape
    b_total = n_seq * seq_len
    b_per_w = b_total // NUM_WORKERS
    w_per_seq = seq_len // b_per_w
    n_chunks = b_per_w // CHUNK
    n_groups = n_chunks // NBUF
    n_vec = CHUNK * D_MODEL // LANES
    col_mask = D_MODEL // LANES - 1

    mesh = plsc.VectorSubcoreMesh(
        core_axis_name="c",
        subcore_axis_name="s",
        num_cores=NUM_CORES,
        num_subcores=NUM_SUBCORES,
    )

    @functools.partial(
        pl.kernel,
        mesh=mesh,
        out_type=jax.ShapeDtypeStruct((n_seq, seq_len, D_MODEL), jnp.float32),
        scratch_types=[
            pltpu.VMEM((b_per_w,), jnp.int32),
            [pltpu.VMEM((CHUNK, D_MODEL), jnp.float32) for _ in range(NBUF)],
            [pltpu.VMEM((CHUNK, D_MODEL), jnp.float32) for _ in range(NBUF)],
            [pltpu.SemaphoreType.DMA for _ in range(NBUF)],
            [pltpu.SemaphoreType.DMA for _ in range(NBUF)],
            pltpu.SemaphoreType.DMA,
        ],
    )
    def k(idx_hbm, table_hbm, out_hbm, idx_v, ins, outs, gsem, wsem, isem):
        wid = lax.axis_index("s") * NUM_CORES + lax.axis_index("c")
        seq_i = wid // w_per_seq
        col0 = (wid % w_per_seq) * b_per_w
        idx_head = b_per_w // 2
        pltpu.sync_copy(
            idx_hbm.at[seq_i, pl.ds(col0, idx_head)], idx_v.at[pl.ds(0, idx_head)]
        )

        def gather_start(g, b):
            pltpu.async_copy(
                table_hbm.at[idx_v.at[pl.ds(g * CHUNK, CHUNK)]], ins[b], gsem[b]
            )

        def gather_wait(g, b):
            pltpu.make_async_copy(
                table_hbm.at[idx_v.at[pl.ds(g * CHUNK, CHUNK)]], ins[b], gsem[b]
            ).wait()

        def write_start(g, b):
            pltpu.async_copy(
                outs[b], out_hbm.at[seq_i, pl.ds(col0 + g * CHUNK, CHUNK)],
                wsem[b],
            )

        def write_wait(g, b):
            pltpu.make_async_copy(
                outs[b], out_hbm.at[seq_i, pl.ds(col0 + g * CHUNK, CHUNK)],
                wsem[b],
            ).wait()

        def scale(b):
            src = ins[b]
            dst = outs[b]

            @plsc.parallel_loop(0, CHUNK * 4, unroll=2)
            def _(i):
                r = lax.shift_right_logical(i, 2)
                q = (i & 3) * (D_MODEL // 4)
                for j in range(D_MODEL // LANES // 4):
                    sl = pl.ds(q + j * LANES, LANES)
                    dst[r, sl] = src[r, sl] * SCALE

        for b in range(NBUF):
            gather_start(b, b)
        tail_src = idx_hbm.at[seq_i, pl.ds(col0 + idx_head, b_per_w - idx_head)]
        tail_dst = idx_v.at[pl.ds(idx_head, b_per_w - idx_head)]
        pltpu.async_copy(tail_src, tail_dst, isem)

        pltpu.make_async_copy(tail_src, tail_dst, isem).wait()

        def body(grp, _):
            for b in range(NBUF):
                g = NBUF * grp + b
                gather_wait(g, b)

                @pl.when(grp > 0)
                def _():
                    write_wait(g - NBUF, b)

                scale(b)
                write_start(g, b)

                @pl.when(grp < n_groups - 1)
                def _():
                    gather_start(g + NBUF, b)

            return 0

        lax.fori_loop(0, n_groups, body, 0)

        for b in range(NBUF):
            write_wait(n_chunks - NBUF + b, b)

    return k(x2d, table)


def kernel(x, table):
    return _sc_embed(x.astype(jnp.int32), table)


# speedup vs baseline: 1.0479x; 1.0479x over previous
import functools
import math

import jax
import jax.numpy as jnp
from jax import lax
from jax.experimental import pallas as pl
from jax.experimental.pallas import tpu as pltpu
from jax.experimental.pallas import tpu_sc as plsc

D_MODEL = 1024
SCALE = math.sqrt(D_MODEL)

NUM_CORES = 2
NUM_SUBCORES = 16
LANES = 16
NUM_WORKERS = NUM_CORES * NUM_SUBCORES

CHUNK = 8
NBUF = 4


def _sc_embed(x2d, table):
    n_seq, seq_len = x2d.shape
    b_total = n_seq * seq_len
    b_per_w = b_total // NUM_WORKERS
    w_per_seq = seq_len // b_per_w
    n_chunks = b_per_w // CHUNK
    n_groups = n_chunks // NBUF
    n_vec = CHUNK * D_MODEL // LANES
    col_mask = D_MODEL // LANES - 1

    mesh = plsc.VectorSubcoreMesh(
        core_axis_name="c",
        subcore_axis_name="s",
        num_cores=NUM_CORES,
        num_subcores=NUM_SUBCORES,
    )

    @functools.partial(
        pl.kernel,
        mesh=mesh,
        out_type=jax.ShapeDtypeStruct((n_seq, seq_len, D_MODEL), jnp.float32),
        scratch_types=[
            pltpu.VMEM((b_per_w,), jnp.int32),
            [pltpu.VMEM((CHUNK, D_MODEL), jnp.float32) for _ in range(NBUF)],
            [pltpu.VMEM((CHUNK, D_MODEL), jnp.float32) for _ in range(NBUF)],
            [pltpu.SemaphoreType.DMA for _ in range(NBUF)],
            [pltpu.SemaphoreType.DMA for _ in range(NBUF)],
            pltpu.SemaphoreType.DMA,
        ],
    )
    def k(idx_hbm, table_hbm, out_hbm, idx_v, ins, outs, gsem, wsem, isem):
        wid = lax.axis_index("s") * NUM_CORES + lax.axis_index("c")
        seq_i = wid // w_per_seq
        col0 = (wid % w_per_seq) * b_per_w
        idx_head = b_per_w // 2
        pltpu.sync_copy(
            idx_hbm.at[seq_i, pl.ds(col0, idx_head)], idx_v.at[pl.ds(0, idx_head)]
        )

        def gather_start(g, b):
            pltpu.async_copy(
                table_hbm.at[idx_v.at[pl.ds(g * CHUNK, CHUNK)]], ins[b], gsem[b]
            )

        def gather_wait(g, b):
            pltpu.make_async_copy(
                table_hbm.at[idx_v.at[pl.ds(g * CHUNK, CHUNK)]], ins[b], gsem[b]
            ).wait()

        def write_start(g, b):
            pltpu.async_copy(
                outs[b], out_hbm.at[seq_i, pl.ds(col0 + g * CHUNK, CHUNK)],
                wsem[b],
            )

        def write_wait(g, b):
            pltpu.make_async_copy(
                outs[b], out_hbm.at[seq_i, pl.ds(col0 + g * CHUNK, CHUNK)],
                wsem[b],
            ).wait()

        def scale(b):
            src = ins[b]
            dst = outs[b]

            @plsc.parallel_loop(0, n_vec, unroll=16)
            def _(i):
                r = lax.shift_right_logical(i, 6)
                sl = pl.ds((i & col_mask) * LANES, LANES)
                dst[r, sl] = src[r, sl] * SCALE

        for b in range(NBUF):
            gather_start(b, b)
        tail_src = idx_hbm.at[seq_i, pl.ds(col0 + idx_head, b_per_w - idx_head)]
        tail_dst = idx_v.at[pl.ds(idx_head, b_per_w - idx_head)]
        pltpu.async_copy(tail_src, tail_dst, isem)

        pltpu.make_async_copy(tail_src, tail_dst, isem).wait()

        def body(grp, _):
            for b in range(NBUF):
                g = NBUF * grp + b
                gather_wait(g, b)

                @pl.when(grp > 0)
                def _():
                    write_wait(g - NBUF, b)

                scale(b)
                write_start(g, b)

                @pl.when(grp < n_groups - 1)
                def _():
                    gather_start(g + NBUF, b)

            return 0

        lax.fori_loop(0, n_groups, body, 0)

        for b in range(NBUF):
            write_wait(n_chunks - NBUF + b, b)

    return k(x2d, table)


def kernel(x, table):
    return _sc_embed(x.astype(jnp.int32), table)


# speedup vs baseline: 1.0510x; 1.0029x over previous
import functools
import math

import jax
import jax.numpy as jnp
from jax import lax
from jax.experimental import pallas as pl
from jax.experimental.pallas import tpu as pltpu
from jax.experimental.pallas import tpu_sc as plsc

D_MODEL = 1024
SCALE = math.sqrt(D_MODEL)

NUM_CORES = 2
NUM_SUBCORES = 16
LANES = 16
NUM_WORKERS = NUM_CORES * NUM_SUBCORES

CHUNK = 8
NBUF = 4


def _sc_embed(x2d, table):
    n_seq, seq_len = x2d.shape
    b_total = n_seq * seq_len
    b_per_w = b_total // NUM_WORKERS
    w_per_seq = seq_len // b_per_w
    n_chunks = b_per_w // CHUNK
    n_groups = n_chunks // NBUF
    n_vec = CHUNK * D_MODEL // LANES
    col_mask = D_MODEL // LANES - 1

    mesh = plsc.VectorSubcoreMesh(
        core_axis_name="c",
        subcore_axis_name="s",
        num_cores=NUM_CORES,
        num_subcores=NUM_SUBCORES,
    )

    @functools.partial(
        pl.kernel,
        mesh=mesh,
        out_type=jax.ShapeDtypeStruct((n_seq, seq_len, D_MODEL), jnp.float32),
        scratch_types=[
            pltpu.VMEM((b_per_w,), jnp.int32),
            [pltpu.VMEM((CHUNK, D_MODEL), jnp.float32) for _ in range(NBUF)],
            [pltpu.VMEM((CHUNK, D_MODEL), jnp.float32) for _ in range(NBUF)],
            [pltpu.SemaphoreType.DMA for _ in range(NBUF)],
            [pltpu.SemaphoreType.DMA for _ in range(NBUF)],
            pltpu.SemaphoreType.DMA,
        ],
    )
    def k(idx_hbm, table_hbm, out_hbm, idx_v, ins, outs, gsem, wsem, isem):
        wid = lax.axis_index("s") * NUM_CORES + lax.axis_index("c")
        seq_i = wid // w_per_seq
        col0 = (wid % w_per_seq) * b_per_w
        idx_head = b_per_w // 2
        pltpu.sync_copy(
            idx_hbm.at[seq_i, pl.ds(col0, idx_head)], idx_v.at[pl.ds(0, idx_head)]
        )

        def gather_start(g, b):
            pltpu.async_copy(
                table_hbm.at[idx_v.at[pl.ds(g * CHUNK, CHUNK)]], ins[b], gsem[b]
            )

        def gather_wait(g, b):
            pltpu.make_async_copy(
                table_hbm.at[idx_v.at[pl.ds(g * CHUNK, CHUNK)]], ins[b], gsem[b]
            ).wait()

        def write_start(g, b):
            pltpu.async_copy(
                outs[b], out_hbm.at[seq_i, pl.ds(col0 + g * CHUNK, CHUNK)],
                wsem[b],
            )

        def write_wait(g, b):
            pltpu.make_async_copy(
                outs[b], out_hbm.at[seq_i, pl.ds(col0 + g * CHUNK, CHUNK)],
                wsem[b],
            ).wait()

        def scale(b):
            src = ins[b]
            dst = outs[b]

            @plsc.parallel_loop(0, n_vec, unroll=8)
            def _(i):
                r = lax.shift_right_logical(i, 6)
                sl = pl.ds((i & col_mask) * LANES, LANES)
                dst[r, sl] = src[r, sl] * SCALE

        for b in range(NBUF):
            gather_start(b, b)
        tail_src = idx_hbm.at[seq_i, pl.ds(col0 + idx_head, b_per_w - idx_head)]
        tail_dst = idx_v.at[pl.ds(idx_head, b_per_w - idx_head)]
        pltpu.async_copy(tail_src, tail_dst, isem)

        pltpu.make_async_copy(tail_src, tail_dst, isem).wait()

        def body(grp, _):
            for b in range(NBUF):
                g = NBUF * grp + b
                gather_wait(g, b)

                @pl.when(grp > 0)
                def _():
                    write_wait(g - NBUF, b)

                scale(b)
                write_start(g, b)

                @pl.when(grp < n_groups - 1)
                def _():
                    gather_start(g + NBUF, b)

            return 0

        lax.fori_loop(0, n_groups, body, 0)

        for b in range(NBUF):
            write_wait(n_chunks - NBUF + b, b)

    return k(x2d, table)


def kernel(x, table):
    return _sc_embed(x.astype(jnp.int32), table)
